# Initial kernel scaffold; baseline (speedup 1.0000x reference)
import jax
import jax.numpy as jnp
from jax.experimental import pallas as pl


def kernel(input_ids, relevance, wte, prompt_embeds):
    raise NotImplementedError("write your pallas kernel here")


# speedup vs baseline: 1.7810x; 1.7810x over previous
import jax
import jax.numpy as jnp
from jax import lax
from jax.experimental import pallas as pl
from jax.experimental.pallas import tpu as pltpu
from jax.experimental.pallas import tpu_sc as plsc

_B = 4
_S = 2048
_H = 768
_N = 16
_NSAMP = _N // _B
_NW = 32
_WPB = _NW // _B
_CHUNK = 64
_NCHUNK_B = _WPB * 4 + 1
_IPAD = 128
_L = 16


def _body(idxp_hbm, rel_hbm, pe_hbm, wte_hbm, out_hbm,
          idx_v, idx_t, rows_v, row_v, rel_v, pe_v, gsem, wsem):
    c = lax.axis_index("c")
    s = lax.axis_index("s")
    wid = s * 2 + c
    b = wid // _WPB
    i = wid % _WPB

    ioff = pl.multiple_of((b * _NCHUNK_B + 4 * i) * _IPAD, _IPAD)
    pltpu.sync_copy(idxp_hbm.at[pl.ds(ioff, 4 * _IPAD)], idx_v)

    for j in range(4):
        buf = rows_v.at[j % 2]
        idx = idx_v.at[pl.ds(j * _IPAD, _CHUNK)]
        pltpu.async_copy(wte_hbm.at[idx], buf, gsem).wait()
        handles = []
        for k in range(_NSAMP):
            roff = pl.multiple_of((4 * i + j) * _CHUNK, _CHUNK)
            dst = out_hbm.at[b + _B * k, pl.ds(roff, _CHUNK)]
            handles.append(pltpu.async_copy(buf, dst, wsem))
        for h in handles:
            h.wait()

    @pl.when(i == _WPB - 1)
    def _():
        toff = pl.multiple_of((b * _NCHUNK_B + 32) * _IPAD, _IPAD)
        pltpu.sync_copy(idxp_hbm.at[pl.ds(toff, _IPAD)], idx_t)
        buf = rows_v.at[0]
        pltpu.async_copy(wte_hbm.at[idx_t.at[pl.ds(0, _CHUNK)]], buf, gsem).wait()
        handles = []
        for k in range(_NSAMP):
            dst = out_hbm.at[b + _B * k, pl.ds(_S, 1)]
            handles.append(pltpu.async_copy(buf.at[pl.ds(0, 1)], dst, wsem))
        for h in handles:
            h.wait()

    @pl.when(i == 0)
    def _():
        pltpu.sync_copy(rel_hbm, rel_v)
        pltpu.sync_copy(pe_hbm, pe_v)
        for k in range(_NSAMP):
            samp = b + _B * k
            r = rel_v[pl.ds(samp * _L, _L)]
            one_m_r = 1.0 - r
            for h in range(_H // _L):
                p0 = pe_v[pl.ds(h * _L, _L)]
                p1 = pe_v[pl.ds(_H + h * _L, _L)]
                row_v[pl.ds(h * _L, _L)] = one_m_r * p0 + r * p1
            pltpu.sync_copy(row_v, out_hbm.at[samp, 0])


_sc_call = pl.kernel(
    _body,
    out_type=jax.ShapeDtypeStruct((_N, _S + 1, _H), jnp.float32),
    mesh=plsc.VectorSubcoreMesh(core_axis_name="c", subcore_axis_name="s"),
    scratch_types=[
        pltpu.VMEM((4 * _IPAD,), jnp.int32),
        pltpu.VMEM((_IPAD,), jnp.int32),
        pltpu.VMEM((2, _CHUNK, _H), jnp.float32),
        pltpu.VMEM((_H,), jnp.float32),
        pltpu.VMEM((_N * _L,), jnp.float32),
        pltpu.VMEM((2 * _H,), jnp.float32),
        pltpu.SemaphoreType.DMA,
        pltpu.SemaphoreType.DMA,
    ],
)


def kernel(input_ids, relevance, wte, prompt_embeds):
    ids = input_ids.astype(jnp.int32)
    shifted = jnp.concatenate(
        [jnp.zeros((_B, 1), jnp.int32), ids], axis=1)
    padded = jnp.pad(shifted, ((0, 0), (0, _NCHUNK_B * _CHUNK - (_S + 1))))
    chunks = padded.reshape(_B, _NCHUNK_B, _CHUNK)
    chunks = jnp.pad(chunks, ((0, 0), (0, 0), (0, _IPAD - _CHUNK)))
    idxp = chunks.reshape(-1)
    rel = jnp.repeat(relevance.astype(jnp.float32), _L)
    pe = prompt_embeds.reshape(-1)
    return _sc_call(idxp, rel, pe, wte)


# speedup vs baseline: 1.8067x; 1.0144x over previous
import jax
import jax.numpy as jnp
from jax import lax
from jax.experimental import pallas as pl
from jax.experimental.pallas import tpu as pltpu
from jax.experimental.pallas import tpu_sc as plsc

_B = 4
_S = 2048
_H = 768
_N = 16
_NSAMP = _N // _B
_NW = 32
_WPB = _NW // _B
_CHUNK = 64
_NCHUNK_B = _WPB * 4 + 1
_IPAD = 128
_L = 16


def _body(idxp_hbm, rel_hbm, pe_hbm, wte_hbm, out_hbm,
          idx_v, idx_t, rows_v, row_v, rel_v, pe_v, gsem, wsem):
    c = lax.axis_index("c")
    s = lax.axis_index("s")
    wid = s * 2 + c
    b = wid // _WPB
    i = wid % _WPB

    ioff = pl.multiple_of((b * _NCHUNK_B + 4 * i) * _IPAD, _IPAD)
    pltpu.sync_copy(idxp_hbm.at[pl.ds(ioff, 4 * _IPAD)], idx_v)

    def _gather(j):
        idx = idx_v.at[pl.ds(j * _IPAD, _CHUNK)]
        return pltpu.async_copy(wte_hbm.at[idx], rows_v.at[j % 2], gsem)

    gh = _gather(0)
    wh = []
    for j in range(4):
        gh.wait()
        if j >= 1:
            for h in wh[j - 1]:
                h.wait()
        if j + 1 < 4:
            gh = _gather(j + 1)
        buf = rows_v.at[j % 2]
        handles = []
        for k in range(_NSAMP):
            roff = pl.multiple_of((4 * i + j) * _CHUNK, _CHUNK)
            dst = out_hbm.at[b + _B * k, pl.ds(roff, _CHUNK)]
            handles.append(pltpu.async_copy(buf, dst, wsem))
        wh.append(handles)
    for h in wh[3]:
        h.wait()

    @pl.when(i == _WPB - 1)
    def _():
        toff = pl.multiple_of((b * _NCHUNK_B + 32) * _IPAD, _IPAD)
        pltpu.sync_copy(idxp_hbm.at[pl.ds(toff, _IPAD)], idx_t)
        buf = rows_v.at[0]
        pltpu.async_copy(wte_hbm.at[idx_t.at[pl.ds(0, _CHUNK)]], buf, gsem).wait()
        handles = []
        for k in range(_NSAMP):
            dst = out_hbm.at[b + _B * k, pl.ds(_S, 1)]
            handles.append(pltpu.async_copy(buf.at[pl.ds(0, 1)], dst, wsem))
        for h in handles:
            h.wait()

    @pl.when(i == 0)
    def _():
        pltpu.sync_copy(rel_hbm, rel_v)
        pltpu.sync_copy(pe_hbm, pe_v)
        for k in range(_NSAMP):
            samp = b + _B * k
            r = rel_v[pl.ds(samp * _L, _L)]
            one_m_r = 1.0 - r
            for h in range(_H // _L):
                p0 = pe_v[pl.ds(h * _L, _L)]
                p1 = pe_v[pl.ds(_H + h * _L, _L)]
                row_v[pl.ds(h * _L, _L)] = one_m_r * p0 + r * p1
            pltpu.sync_copy(row_v, out_hbm.at[samp, 0])


_sc_call = pl.kernel(
    _body,
    out_type=jax.ShapeDtypeStruct((_N, _S + 1, _H), jnp.float32),
    mesh=plsc.VectorSubcoreMesh(core_axis_name="c", subcore_axis_name="s"),
    scratch_types=[
        pltpu.VMEM((4 * _IPAD,), jnp.int32),
        pltpu.VMEM((_IPAD,), jnp.int32),
        pltpu.VMEM((2, _CHUNK, _H), jnp.float32),
        pltpu.VMEM((_H,), jnp.float32),
        pltpu.VMEM((_N * _L,), jnp.float32),
        pltpu.VMEM((2 * _H,), jnp.float32),
        pltpu.SemaphoreType.DMA,
        pltpu.SemaphoreType.DMA,
    ],
)


def kernel(input_ids, relevance, wte, prompt_embeds):
    ids = input_ids.astype(jnp.int32)
    shifted = jnp.concatenate(
        [jnp.zeros((_B, 1), jnp.int32), ids], axis=1)
    padded = jnp.pad(shifted, ((0, 0), (0, _NCHUNK_B * _CHUNK - (_S + 1))))
    chunks = padded.reshape(_B, _NCHUNK_B, _CHUNK)
    chunks = jnp.pad(chunks, ((0, 0), (0, 0), (0, _IPAD - _CHUNK)))
    idxp = chunks.reshape(-1)
    rel = jnp.repeat(relevance.astype(jnp.float32), _L)
    pe = prompt_embeds.reshape(-1)
    return _sc_call(idxp, rel, pe, wte)


# speedup vs baseline: 2.4205x; 1.3397x over previous
import jax
import jax.numpy as jnp
from jax import lax
from jax.experimental import pallas as pl
from jax.experimental.pallas import tpu as pltpu
from jax.experimental.pallas import tpu_sc as plsc

_B = 4
_S = 2048
_H = 768
_N = 16
_NSAMP = _N // _B
_NW = 32
_TPW = _S // _NW
_CTOK = 4
_CROWS = _CTOK * _N
_NCHUNK = _TPW // _CTOK
_L = 16


def _body(idxp_hbm, rel_hbm, pe_hbm, wte_hbm, out_hbm,
          idx_v, rows_v, blend_v, rel_v, pe_v, gsem, wsem):
    c = lax.axis_index("c")
    s = lax.axis_index("s")
    wid = s * 2 + c

    ioff = pl.multiple_of(wid * (_TPW * _N), 8)
    pltpu.sync_copy(idxp_hbm.at[pl.ds(ioff, _TPW * _N)], idx_v)

    def _gather(j):
        idx = idx_v.at[pl.ds(j * _CROWS, _CROWS)]
        return pltpu.async_copy(wte_hbm.at[idx], rows_v.at[j % 2], gsem)

    gh = _gather(0)
    wh = []
    for j in range(_NCHUNK):
        gh.wait()
        if j >= 1:
            wh[j - 1].wait()
        if j + 1 < _NCHUNK:
            gh = _gather(j + 1)
        roff = pl.multiple_of(_N + wid * (_TPW * _N) + j * _CROWS, _N)
        dst = out_hbm.at[pl.ds(roff, _CROWS)]
        wh.append(pltpu.async_copy(rows_v.at[j % 2], dst, wsem))
    wh[_NCHUNK - 1].wait()

    @pl.when(wid == 0)
    def _():
        pltpu.sync_copy(rel_hbm, rel_v)
        pltpu.sync_copy(pe_hbm, pe_v)
        for samp in range(_N):
            r = rel_v[pl.ds(samp * _L, _L)]
            one_m_r = 1.0 - r
            row = blend_v.at[samp]
            for h in range(_H // _L):
                p0 = pe_v[pl.ds(h * _L, _L)]
                p1 = pe_v[pl.ds(_H + h * _L, _L)]
                row[pl.ds(h * _L, _L)] = one_m_r * p0 + r * p1
        pltpu.sync_copy(blend_v, out_hbm.at[pl.ds(0, _N)])


_sc_call = pl.kernel(
    _body,
    out_type=jax.ShapeDtypeStruct(((_S + 1) * _N, _H), jnp.float32),
    mesh=plsc.VectorSubcoreMesh(core_axis_name="c", subcore_axis_name="s"),
    scratch_types=[
        pltpu.VMEM((_TPW * _N,), jnp.int32),
        pltpu.VMEM((2, _CROWS, _H), jnp.float32),
        pltpu.VMEM((_N, _H), jnp.float32),
        pltpu.VMEM((_N * _L,), jnp.float32),
        pltpu.VMEM((2 * _H,), jnp.float32),
        pltpu.SemaphoreType.DMA,
        pltpu.SemaphoreType.DMA,
    ],
)


def kernel(input_ids, relevance, wte, prompt_embeds):
    ids = input_ids.astype(jnp.int32)
    idxp = jnp.tile(ids.T, (1, _NSAMP)).reshape(-1)
    rel = jnp.repeat(relevance.astype(jnp.float32), _L)
    pe = prompt_embeds.reshape(-1)
    out2d = _sc_call(idxp, rel, pe, wte)
    return out2d.reshape(_S + 1, _N, _H).transpose(1, 0, 2)


# speedup vs baseline: 3.3121x; 1.3683x over previous
import jax
import jax.numpy as jnp
from jax import lax
from jax.experimental import pallas as pl
from jax.experimental.pallas import tpu as pltpu
from jax.experimental.pallas import tpu_sc as plsc

_B = 4
_S = 2048
_H = 768
_N = 16
_NSAMP = _N // _B
_NW = 32
_TPW = _S // _NW
_G = 8
_CTOK = 8
_NCHUNK = _TPW // _CTOK
_L = 16


def _body(idxp_hbm, rel_hbm, pe_hbm, wte_hbm, out_hbm,
          idx_v, rows_v, blend_v, rel_v, pe_v, gsem, wsem):
    c = lax.axis_index("c")
    s = lax.axis_index("s")
    wid = s * 2 + c

    ioff = pl.multiple_of(wid * (_TPW * _G), 8)
    pltpu.sync_copy(idxp_hbm.at[pl.ds(ioff, _TPW * _G)], idx_v)

    def _gather(j):
        hs = []
        for tt in range(_CTOK):
            idx = idx_v.at[pl.ds((j * _CTOK + tt) * _G, _G)]
            hs.append(pltpu.async_copy(wte_hbm.at[idx],
                                       rows_v.at[j % 2, tt], gsem))
        return hs

    gh = _gather(0)
    wh = []
    for j in range(_NCHUNK):
        for h in gh:
            h.wait()
        if j >= 1:
            for h in wh[j - 1]:
                h.wait()
        if j + 1 < _NCHUNK:
            gh = _gather(j + 1)
        t0 = pl.multiple_of(1 + wid * _TPW + j * _CTOK, 1)
        buf = rows_v.at[j % 2]
        wh.append([
            pltpu.async_copy(buf, out_hbm.at[pl.ds(t0, _CTOK), k], wsem)
            for k in range(2)
        ])
    for h in wh[_NCHUNK - 1]:
        h.wait()

    @pl.when(wid == 0)
    def _():
        pltpu.sync_copy(rel_hbm, rel_v)
        pltpu.sync_copy(pe_hbm, pe_v)
        for samp in range(_N):
            r = rel_v[pl.ds(samp * _L, _L)]
            one_m_r = 1.0 - r
            row = blend_v.at[samp // _G, samp % _G]
            for h in range(_H // _L):
                p0 = pe_v[pl.ds(h * _L, _L)]
                p1 = pe_v[pl.ds(_H + h * _L, _L)]
                row[pl.ds(h * _L, _L)] = one_m_r * p0 + r * p1
        pltpu.sync_copy(blend_v, out_hbm.at[0])


_sc_call = pl.kernel(
    _body,
    out_type=jax.ShapeDtypeStruct((_S + 1, 2, _G, _H), jnp.float32),
    mesh=plsc.VectorSubcoreMesh(core_axis_name="c", subcore_axis_name="s"),
    scratch_types=[
        pltpu.VMEM((_TPW * _G,), jnp.int32),
        pltpu.VMEM((2, _CTOK, _G, _H), jnp.float32),
        pltpu.VMEM((2, _G, _H), jnp.float32),
        pltpu.VMEM((_N * _L,), jnp.float32),
        pltpu.VMEM((2 * _H,), jnp.float32),
        pltpu.SemaphoreType.DMA,
        pltpu.SemaphoreType.DMA,
    ],
)


def kernel(input_ids, relevance, wte, prompt_embeds):
    ids = input_ids.astype(jnp.int32)
    idxp = jnp.tile(ids.T, (1, _G // _B)).reshape(-1)
    rel = jnp.repeat(relevance.astype(jnp.float32), _L)
    pe = prompt_embeds.reshape(-1)
    out = _sc_call(idxp, rel, pe, wte)
    return out.reshape(_S + 1, _N, _H).transpose(1, 0, 2)


# speedup vs baseline: 3.3256x; 1.0041x over previous
import jax
import jax.numpy as jnp
from jax import lax
from jax.experimental import pallas as pl
from jax.experimental.pallas import tpu as pltpu
from jax.experimental.pallas import tpu_sc as plsc

_B = 4
_S = 2048
_H = 768
_N = 16
_NSAMP = _N // _B
_NW = 32
_TPW = _S // _NW
_G = 8
_CTOK = 8
_NCHUNK = _TPW // _CTOK
_L = 16


def _body(idxp_hbm, rel_hbm, pe_hbm, wte_hbm, out_hbm,
          idx_v, rows_v, blend_v, rel_v, pe_v, gsem, wsem):
    c = lax.axis_index("c")
    s = lax.axis_index("s")
    wid = s * 2 + c

    ioff = pl.multiple_of(wid * (_TPW * _G), 8)
    pltpu.sync_copy(idxp_hbm.at[pl.ds(ioff, _TPW * _G)], idx_v)

    def _gather(j):
        hs = []
        for tt in range(_CTOK):
            idx = idx_v.at[pl.ds((j * _CTOK + tt) * _G, _G)]
            hs.append(pltpu.async_copy(wte_hbm.at[idx],
                                       rows_v.at[j % 2, tt], gsem))
        return hs

    gh = _gather(0)
    wh = []
    for j in range(_NCHUNK):
        for h in gh:
            h.wait()
        if j >= 1:
            for h in wh[j - 1]:
                h.wait()
        if j + 1 < _NCHUNK:
            gh = _gather(j + 1)
        t0 = pl.multiple_of(1 + wid * _TPW + j * _CTOK, 1)
        buf = rows_v.at[j % 2]
        wh.append([
            pltpu.async_copy(buf, out_hbm.at[pl.ds(t0, _CTOK), k], wsem)
            for k in range(2)
        ])
    for h in wh[_NCHUNK - 1]:
        h.wait()

    @pl.when(wid == 0)
    def _():
        pltpu.sync_copy(rel_hbm, rel_v)
        pltpu.sync_copy(pe_hbm, pe_v)
        for samp in range(_N):
            r = rel_v[pl.ds(samp * _L, _L)]
            one_m_r = 1.0 - r
            row = blend_v.at[samp // _G, samp % _G]
            for h in range(_H // _L):
                p0 = pe_v[pl.ds(h * _L, _L)]
                p1 = pe_v[pl.ds(_H + h * _L, _L)]
                row[pl.ds(h * _L, _L)] = one_m_r * p0 + r * p1
        pltpu.sync_copy(blend_v, out_hbm.at[0])


_sc_call = pl.kernel(
    _body,
    out_type=jax.ShapeDtypeStruct((_S + 1, 2, _G, _H), jnp.float32),
    mesh=plsc.VectorSubcoreMesh(core_axis_name="c", subcore_axis_name="s"),
    scratch_types=[
        pltpu.VMEM((_TPW * _G,), jnp.int32),
        pltpu.VMEM((2, _CTOK, _G, _H), jnp.float32),
        pltpu.VMEM((2, _G, _H), jnp.float32),
        pltpu.VMEM((_N * _L,), jnp.float32),
        pltpu.VMEM((2 * _H,), jnp.float32),
        pltpu.SemaphoreType.DMA,
        pltpu.SemaphoreType.DMA,
    ],
)


def kernel(input_ids, relevance, wte, prompt_embeds):
    ids = input_ids.astype(jnp.int32)
    idxp = jnp.broadcast_to(ids.T[:, None, :],
                            (_S, _G // _B, _B)).reshape(-1)
    rel = jnp.repeat(relevance.astype(jnp.float32), _L)
    pe = prompt_embeds.reshape(-1)
    out = _sc_call(idxp, rel, pe, wte)
    return out.reshape(_S + 1, _N, _H).transpose(1, 0, 2)
